# baseline (speedup 1.0000x reference)
import functools

import jax
import jax.numpy as jnp
from jax import lax
from jax.experimental import pallas as pl
from jax.experimental.pallas import tpu as pltpu
from jax.experimental.pallas import tpu_sc as plsc

N = 50000
E = 800000
CIN = 128
HID = 64
HHALF = HID // 2
NGRAPH = 128
NCLS = 7
NC = 2
NS = 16
CHUNK = 128

CONV_CHUNKS = 400
E_CONV = NS * CONV_CHUNKS * CHUNK
IDX_G = 16
IDX_GROUPS = CONV_CHUNKS // IDX_G
DEG_CHUNKS = 196
E_DEG = NC * NS * DEG_CHUNKS * CHUNK

NROW = 50048
SINK = N
TILE_ROWS = NROW // NS

R = 1000
NBLK = N // R

_mesh = plsc.VectorSubcoreMesh(core_axis_name="c", subcore_axis_name="s")
_sc_params = pltpu.CompilerParams(use_tc_tiling_on_sc=False)


DEG_LAG = 8


def _deg_body(dst_hbm, zeros_hbm, ones_hbm, out0_hbm, out1_hbm,
              acc, idxbuf, onesbuf, sem):
    cid = lax.axis_index("c")
    sid = lax.axis_index("s")
    wid = sid * NC + cid
    pltpu.sync_copy(ones_hbm, onesbuf)
    pltpu.sync_copy(dst_hbm.at[wid], idxbuf)
    pltpu.sync_copy(zeros_hbm, acc.at[pl.ds(sid * TILE_ROWS, TILE_ROWS)])
    plsc.subcore_barrier()

    def scat(j):
        return pltpu.async_copy(onesbuf, acc.at[idxbuf.at[j]], sem, add=True)

    def wait_one():
        pltpu.make_async_copy(onesbuf, acc.at[idxbuf.at[0]], sem).wait()

    for j in range(DEG_LAG):
        scat(j)

    def body(j, carry):
        wait_one()
        scat(j)
        return carry

    lax.fori_loop(DEG_LAG, DEG_CHUNKS, body, 0)
    for _ in range(DEG_LAG):
        wait_one()
    plsc.subcore_barrier()

    def out(out_hbm):
        sl = pl.ds(sid * TILE_ROWS, TILE_ROWS)
        pltpu.sync_copy(acc.at[sl], out_hbm.at[sl])

    @pl.when(cid == 0)
    def _():
        out(out0_hbm)

    @pl.when(cid == 1)
    def _():
        out(out1_hbm)


def _deg_call(dst_d, zeros_init, ones_rows):
    return pl.kernel(
        _deg_body,
        out_type=(
            jax.ShapeDtypeStruct((NROW, HHALF), jnp.float32),
            jax.ShapeDtypeStruct((NROW, HHALF), jnp.float32),
        ),
        mesh=_mesh,
        compiler_params=_sc_params,
        scratch_types=[
            pltpu.VMEM_SHARED((NROW, HHALF), jnp.float32),
            pltpu.VMEM((DEG_CHUNKS, CHUNK), jnp.int32),
            pltpu.VMEM((CHUNK, HHALF), jnp.float32),
            pltpu.SemaphoreType.DMA,
        ],
    )(dst_d, zeros_init, ones_rows)


NBUF = 4
LOOK = 2


def _conv_body(g0_hbm, g1_hbm, src_hbm, dst_hbm, out0_hbm, out1_hbm,
               acc, srcbuf, dstbuf, rows,
               sem_i, sg0, sg1, sg2, sg3, ss0, ss1, ss2, ss3):
    cid = lax.axis_index("c")
    sid = lax.axis_index("s")
    sem_g = (sg0, sg1, sg2, sg3)
    sem_s = (ss0, ss1, ss2, ss3)

    def init(g_hbm):
        sl = pl.ds(sid * TILE_ROWS, TILE_ROWS)
        pltpu.sync_copy(g_hbm.at[sl], acc.at[sl])

    @pl.when(cid == 0)
    def _():
        init(g0_hbm)

    @pl.when(cid == 1)
    def _():
        init(g1_hbm)

    plsc.subcore_barrier()

    def edges(g_hbm):
        def wait_gather(b):
            pltpu.make_async_copy(g_hbm.at[srcbuf.at[0].at[0]], rows.at[b],
                                  sem_g[b]).wait()

        def wait_scatter(b):
            pass

        pltpu.sync_copy(src_hbm.at[sid].at[pl.ds(0, IDX_G)], srcbuf.at[0])
        pltpu.sync_copy(dst_hbm.at[sid].at[pl.ds(0, IDX_G)], dstbuf.at[0])
        for j in range(LOOK):
            pltpu.async_copy(g_hbm.at[srcbuf.at[0].at[j]], rows.at[j], sem_g[j])

        def group_body(gi, first):
            ib = gi % 2 if first else lax.rem(gi, 2)
            nib = 1 - ib
            ng = jnp.minimum(gi + 1, IDX_GROUPS - 1)
            nsl = pl.ds(ng * IDX_G, IDX_G)
            d_a = pltpu.async_copy(src_hbm.at[sid].at[nsl], srcbuf.at[nib], sem_i)
            d_b = pltpu.async_copy(dst_hbm.at[sid].at[nsl], dstbuf.at[nib], sem_i)
            sb = srcbuf.at[ib]
            db = dstbuf.at[ib]
            nsb = srcbuf.at[nib]

            def do_gather(j, b, buf):
                return pltpu.async_copy(g_hbm.at[buf.at[j]], rows.at[b], sem_g[b])

            class _NoOp:
                def wait(self):
                    pass

            def do_scatter(j, b):
                return _NoOp()

            gat = {}
            sca = {}
            for j in range(IDX_G):
                b = j % NBUF
                t = j + LOOK
                if t < IDX_G:
                    if j - LOOK >= 0:
                        sca[j - LOOK].wait()
                    elif not first:
                        wait_scatter(t % NBUF)
                    gat[t] = do_gather(t, t % NBUF, sb)
                else:
                    sca[j - LOOK].wait()
                    if t == IDX_G:
                        d_a.wait()
                        d_b.wait()
                    tn = t - IDX_G
                    do_gather(tn, tn % NBUF, nsb)
                if j in gat:
                    gat[j].wait()
                else:
                    wait_gather(b)
                sca[j] = do_scatter(j, b)

        group_body(0, True)

        def group(gi, carry):
            group_body(gi, False)
            return carry

        lax.fori_loop(1, IDX_GROUPS, group, 0)
        for j in range(LOOK):
            wait_gather(j)
        for j in range(IDX_G - LOOK, IDX_G):
            wait_scatter(j % NBUF)

    @pl.when(cid == 0)
    def _():
        edges(g0_hbm)

    @pl.when(cid == 1)
    def _():
        edges(g1_hbm)

    plsc.subcore_barrier()

    def out(out_hbm):
        sl = pl.ds(sid * TILE_ROWS, TILE_ROWS)
        pltpu.sync_copy(acc.at[sl], out_hbm.at[sl])

    @pl.when(cid == 0)
    def _():
        out(out0_hbm)

    @pl.when(cid == 1)
    def _():
        out(out1_hbm)


def _conv_call(g0, g1, src_c, dst_c):
    return pl.kernel(
        _conv_body,
        out_type=(
            jax.ShapeDtypeStruct((NROW, HHALF), jnp.float32),
            jax.ShapeDtypeStruct((NROW, HHALF), jnp.float32),
        ),
        mesh=_mesh,
        compiler_params=_sc_params,
        scratch_types=[
            pltpu.VMEM_SHARED((NROW, HHALF), jnp.float32),
            pltpu.VMEM((2, IDX_G, CHUNK), jnp.int32),
            pltpu.VMEM((2, IDX_G, CHUNK), jnp.int32),
            pltpu.VMEM((NBUF, CHUNK, HHALF), jnp.float32),
        ] + [pltpu.SemaphoreType.DMA] * 9,
    )(g0, g1, src_c, dst_c)


NROW4 = NROW * HHALF // 128
RP = 256
NBLK = -(-NROW4 // RP)


def _dot(a, b):
    return lax.dot_general(a, b, (((1,), (0,)), ((), ())),
                           precision=lax.Precision.HIGHEST,
                           preferred_element_type=jnp.float32)


def _prep_body(x4_ref, dg0_ref, dg1_ref, w0_ref, w1_ref, g0_ref, g1_ref, dv_ref):
    dv = lax.rsqrt(dg0_ref[...] + dg1_ref[...] + 1.0)
    dv_ref[...] = dv
    g0_ref[...] = _dot(x4_ref[...], w0_ref[...]) * dv
    g1_ref[...] = _dot(x4_ref[...], w1_ref[...]) * dv


def _prep_call(x4, dg0, dg1, w1bd0, w1bd1):
    return pl.pallas_call(
        _prep_body,
        grid=(NBLK,),
        in_specs=[
            pl.BlockSpec((RP, 4 * CIN), lambda i: (i, 0)),
            pl.BlockSpec((RP, 128), lambda i: (i, 0)),
            pl.BlockSpec((RP, 128), lambda i: (i, 0)),
            pl.BlockSpec((4 * CIN, 128), lambda i: (0, 0)),
            pl.BlockSpec((4 * CIN, 128), lambda i: (0, 0)),
        ],
        out_specs=[pl.BlockSpec((RP, 128), lambda i: (i, 0))] * 3,
        out_shape=[jax.ShapeDtypeStruct((NROW4, 128), jnp.float32)] * 3,
    )(x4, dg0, dg1, w1bd0, w1bd1)


def _mid_body(a0_ref, a1_ref, dv_ref, b0_ref, b1_ref,
              w00_ref, w10_ref, w01_ref, w11_ref, g0_ref, g1_ref):
    dv = dv_ref[...]
    h0 = jnp.maximum(a0_ref[...] * dv + b0_ref[...], 0.0)
    h1 = jnp.maximum(a1_ref[...] * dv + b1_ref[...], 0.0)
    g0_ref[...] = (_dot(h0, w00_ref[...]) + _dot(h1, w10_ref[...])) * dv
    g1_ref[...] = (_dot(h0, w01_ref[...]) + _dot(h1, w11_ref[...])) * dv


def _mid_call(a0p, a1p, dinvp, b0p, b1p, wbds):
    return pl.pallas_call(
        _mid_body,
        grid=(NBLK,),
        in_specs=[
            pl.BlockSpec((RP, 128), lambda i: (i, 0)),
            pl.BlockSpec((RP, 128), lambda i: (i, 0)),
            pl.BlockSpec((RP, 128), lambda i: (i, 0)),
            pl.BlockSpec((1, 128), lambda i: (0, 0)),
            pl.BlockSpec((1, 128), lambda i: (0, 0)),
        ] + [pl.BlockSpec((128, 128), lambda i: (0, 0))] * 4,
        out_specs=[pl.BlockSpec((RP, 128), lambda i: (i, 0))] * 2,
        out_shape=[jax.ShapeDtypeStruct((NROW4, 128), jnp.float32)] * 2,
    )(a0p, a1p, dinvp, b0p, b1p, *wbds)


def _final_body(a0_ref, a1_ref, dv_ref, b0_ref, b1_ref, batchq_ref,
                wf1_ref, bf1_ref, wf2_ref, bf2_ref, out_ref, pooled, cnt):
    i = pl.program_id(0)

    @pl.when(i == 0)
    def _():
        pooled[...] = jnp.zeros_like(pooled)
        cnt[...] = jnp.zeros_like(cnt)

    dv = dv_ref[...]
    h0 = jnp.maximum(a0_ref[...] * dv + b0_ref[...], 0.0)
    h1 = jnp.maximum(a1_ref[...] * dv + b1_ref[...], 0.0)
    rid = i * RP + lax.broadcasted_iota(jnp.int32, (RP, 1), 0)
    ok = rid < NROW4
    h0 = jnp.where(ok, h0, 0.0)
    h1 = jnp.where(ok, h1, 0.0)
    p0 = jnp.zeros((NGRAPH, HHALF), jnp.float32)
    p1 = jnp.zeros((NGRAPH, HHALF), jnp.float32)
    c8 = jnp.zeros((NGRAPH, 8), jnp.float32)
    for q in range(4):
        bq = batchq_ref[0, q, :]
        oh = (lax.broadcasted_iota(jnp.int32, (NGRAPH, RP), 0)
              == bq[None, :]).astype(jnp.float32)
        p0 += _dot(oh, h0[:, 32 * q:32 * q + 32])
        p1 += _dot(oh, h1[:, 32 * q:32 * q + 32])
        c8 += _dot(oh, jnp.ones((RP, 8), jnp.float32))
    pooled[...] += jnp.concatenate([p0, p1], axis=1)
    cnt[...] += c8

    @pl.when(i == NBLK - 1)
    def _():
        pm = pooled[...] / jnp.maximum(cnt[...][:, :1], 1.0)
        z = jnp.maximum(_dot(pm, wf1_ref[...]) + bf1_ref[...], 0.0)
        out_ref[...] = _dot(z, wf2_ref[...]) + bf2_ref[...]


def _final_call(a0p, a1p, dinvp, b0p, b1p, batchq, Wf1, bf1r, Wf2p, bf2p):
    return pl.pallas_call(
        _final_body,
        grid=(NBLK,),
        in_specs=[
            pl.BlockSpec((RP, 128), lambda i: (i, 0)),
            pl.BlockSpec((RP, 128), lambda i: (i, 0)),
            pl.BlockSpec((RP, 128), lambda i: (i, 0)),
            pl.BlockSpec((1, 128), lambda i: (0, 0)),
            pl.BlockSpec((1, 128), lambda i: (0, 0)),
            pl.BlockSpec((1, 4, RP), lambda i: (i, 0, 0)),
            pl.BlockSpec((HID, HID), lambda i: (0, 0)),
            pl.BlockSpec((1, HID), lambda i: (0, 0)),
            pl.BlockSpec((HID, 128), lambda i: (0, 0)),
            pl.BlockSpec((1, 128), lambda i: (0, 0)),
        ],
        out_specs=pl.BlockSpec((NGRAPH, 128), lambda i: (0, 0)),
        out_shape=jax.ShapeDtypeStruct((NGRAPH, 128), jnp.float32),
        scratch_shapes=[
            pltpu.VMEM((NGRAPH, HID), jnp.float32),
            pltpu.VMEM((NGRAPH, 8), jnp.float32),
        ],
    )(a0p, a1p, dinvp, b0p, b1p, batchq, Wf1, bf1r, Wf2p, bf2p)


def _bd4(m):
    return jnp.kron(jnp.eye(4, dtype=m.dtype), m)


def _packb(b_half):
    return jnp.tile(b_half, 4).reshape(1, 128)


def kernel(x, edge_index, batch, W1, b1, W2, b2, W3, b3, Wf1, bf1, Wf2, bf2):
    i32 = jnp.int32
    src = edge_index[0]
    dst = edge_index[1]
    pad_src = (jnp.arange(E_CONV - E, dtype=i32) * 641) % N
    pad_dst = SINK + jnp.arange(E_CONV - E, dtype=i32) % (NROW - N)
    src_c = jnp.concatenate([src, pad_src]).reshape(NS, CONV_CHUNKS, CHUNK)
    dst_c = jnp.concatenate([dst, pad_dst]).reshape(NS, CONV_CHUNKS, CHUNK)
    pad_dd = SINK + jnp.arange(E_DEG - E, dtype=i32) % (NROW - N)
    dst_d = jnp.concatenate([dst, pad_dd]).reshape(NC * NS, DEG_CHUNKS, CHUNK)
    zeros_init = jnp.zeros((TILE_ROWS, HHALF), jnp.float32)
    ones_rows = jnp.ones((CHUNK, HHALF), jnp.float32)
    x4 = jnp.pad(x, ((0, NROW - N), (0, 0))).reshape(NROW4, 4 * CIN)
    batchq = (jnp.concatenate([batch, jnp.full((NBLK * 4 * RP - N,), -1, i32)])
              .reshape(NBLK, RP, 4).transpose(0, 2, 1))
    w1bd0 = _bd4(W1[:, :HHALF])
    w1bd1 = _bd4(W1[:, HHALF:])

    def mid_wbds(W):
        return (_bd4(W[:HHALF, :HHALF]), _bd4(W[HHALF:, :HHALF]),
                _bd4(W[:HHALF, HHALF:]), _bd4(W[HHALF:, HHALF:]))

    bf1r = bf1.reshape(1, HID)
    Wf2p = jnp.pad(Wf2, ((0, 0), (0, 128 - NCLS)))
    bf2p = jnp.pad(bf2, (0, 128 - NCLS)).reshape(1, 128)

    dg0, dg1 = _deg_call(dst_d, zeros_init, ones_rows)
    g0p, g1p, dinvp = _prep_call(x4, dg0.reshape(NROW4, 128),
                                 dg1.reshape(NROW4, 128), w1bd0, w1bd1)

    def conv(g0p_, g1p_):
        a0, a1 = _conv_call(g0p_.reshape(NROW, HHALF), g1p_.reshape(NROW, HHALF),
                            src_c, dst_c)
        return a0.reshape(NROW4, 128), a1.reshape(NROW4, 128)

    a0p, a1p = conv(g0p, g1p)
    g0p, g1p = _mid_call(a0p, a1p, dinvp, _packb(b1[:HHALF]), _packb(b1[HHALF:]),
                         mid_wbds(W2))
    a0p, a1p = conv(g0p, g1p)
    g0p, g1p = _mid_call(a0p, a1p, dinvp, _packb(b2[:HHALF]), _packb(b2[HHALF:]),
                         mid_wbds(W3))
    a0p, a1p = conv(g0p, g1p)
    outp = _final_call(a0p, a1p, dinvp, _packb(b3[:HHALF]), _packb(b3[HHALF:]),
                       batchq, Wf1, bf1r, Wf2p, bf2p)
    return outp[:, :NCLS]


# speedup vs baseline: 1.0199x; 1.0199x over previous
import functools

import jax
import jax.numpy as jnp
from jax import lax
from jax.experimental import pallas as pl
from jax.experimental.pallas import tpu as pltpu
from jax.experimental.pallas import tpu_sc as plsc

N = 50000
E = 800000
CIN = 128
HID = 64
HHALF = HID // 2
NGRAPH = 128
NCLS = 7
NC = 2
NS = 16
CHUNK = 128

CONV_CHUNKS = 400
E_CONV = NS * CONV_CHUNKS * CHUNK
IDX_G = 16
IDX_GROUPS = CONV_CHUNKS // IDX_G
DEG_CHUNKS = 196
E_DEG = NC * NS * DEG_CHUNKS * CHUNK

NROW = 50048
SINK = N
TILE_ROWS = NROW // NS

R = 1000
NBLK = N // R

_mesh = plsc.VectorSubcoreMesh(core_axis_name="c", subcore_axis_name="s")
_sc_params = pltpu.CompilerParams(use_tc_tiling_on_sc=False)


DEG_LAG = 8


def _deg_body(dst_hbm, zeros_hbm, ones_hbm, out0_hbm, out1_hbm,
              acc, idxbuf, onesbuf, sem):
    cid = lax.axis_index("c")
    sid = lax.axis_index("s")
    wid = sid * NC + cid
    pltpu.sync_copy(ones_hbm, onesbuf)
    pltpu.sync_copy(dst_hbm.at[wid], idxbuf)
    pltpu.sync_copy(zeros_hbm, acc.at[pl.ds(sid * TILE_ROWS, TILE_ROWS)])
    plsc.subcore_barrier()

    def scat(j):
        return pltpu.async_copy(onesbuf, acc.at[idxbuf.at[j]], sem, add=True)

    def wait_one():
        pltpu.make_async_copy(onesbuf, acc.at[idxbuf.at[0]], sem).wait()

    for j in range(DEG_LAG):
        scat(j)

    def body(j, carry):
        wait_one()
        scat(j)
        return carry

    lax.fori_loop(DEG_LAG, DEG_CHUNKS, body, 0)
    for _ in range(DEG_LAG):
        wait_one()
    plsc.subcore_barrier()

    def out(out_hbm):
        sl = pl.ds(sid * TILE_ROWS, TILE_ROWS)
        pltpu.sync_copy(acc.at[sl], out_hbm.at[sl])

    @pl.when(cid == 0)
    def _():
        out(out0_hbm)

    @pl.when(cid == 1)
    def _():
        out(out1_hbm)


def _deg_call(dst_d, zeros_init, ones_rows):
    return pl.kernel(
        _deg_body,
        out_type=(
            jax.ShapeDtypeStruct((NROW, HHALF), jnp.float32),
            jax.ShapeDtypeStruct((NROW, HHALF), jnp.float32),
        ),
        mesh=_mesh,
        compiler_params=_sc_params,
        scratch_types=[
            pltpu.VMEM_SHARED((NROW, HHALF), jnp.float32),
            pltpu.VMEM((DEG_CHUNKS, CHUNK), jnp.int32),
            pltpu.VMEM((CHUNK, HHALF), jnp.float32),
            pltpu.SemaphoreType.DMA,
        ],
    )(dst_d, zeros_init, ones_rows)


NBUF = 4
LOOK = 2


def _conv_body(g0_hbm, g1_hbm, src_hbm, dst_hbm, out0_hbm, out1_hbm,
               acc, srcbuf, dstbuf, rows,
               sem_i, sg0, sg1, sg2, sg3, ss0, ss1, ss2, ss3):
    cid = lax.axis_index("c")
    sid = lax.axis_index("s")
    sem_g = (sg0, sg1, sg2, sg3)
    sem_s = (ss0, ss1, ss2, ss3)

    def init(g_hbm):
        sl = pl.ds(sid * TILE_ROWS, TILE_ROWS)
        pltpu.sync_copy(g_hbm.at[sl], acc.at[sl])

    @pl.when(cid == 0)
    def _():
        init(g0_hbm)

    @pl.when(cid == 1)
    def _():
        init(g1_hbm)

    plsc.subcore_barrier()

    def edges(g_hbm):
        def wait_gather(b):
            pltpu.make_async_copy(g_hbm.at[srcbuf.at[0].at[0]], rows.at[b],
                                  sem_g[b]).wait()

        def wait_scatter(b):
            pltpu.make_async_copy(rows.at[b], acc.at[dstbuf.at[0].at[0]],
                                  sem_s[b]).wait()

        pltpu.sync_copy(src_hbm.at[sid].at[pl.ds(0, IDX_G)], srcbuf.at[0])
        pltpu.sync_copy(dst_hbm.at[sid].at[pl.ds(0, IDX_G)], dstbuf.at[0])
        for j in range(LOOK):
            pltpu.async_copy(g_hbm.at[srcbuf.at[0].at[j]], rows.at[j], sem_g[j])

        def group_body(gi, first):
            ib = gi % 2 if first else lax.rem(gi, 2)
            nib = 1 - ib
            ng = jnp.minimum(gi + 1, IDX_GROUPS - 1)
            nsl = pl.ds(ng * IDX_G, IDX_G)
            d_a = pltpu.async_copy(src_hbm.at[sid].at[nsl], srcbuf.at[nib], sem_i)
            d_b = pltpu.async_copy(dst_hbm.at[sid].at[nsl], dstbuf.at[nib], sem_i)
            sb = srcbuf.at[ib]
            db = dstbuf.at[ib]
            nsb = srcbuf.at[nib]

            def do_gather(j, b, buf):
                return pltpu.async_copy(g_hbm.at[buf.at[j]], rows.at[b], sem_g[b])

            def do_scatter(j, b):
                return pltpu.async_copy(rows.at[b], acc.at[db.at[j]], sem_s[b],
                                        add=True)

            gat = {}
            sca = {}
            for j in range(IDX_G):
                b = j % NBUF
                t = j + LOOK
                if t < IDX_G:
                    if j - LOOK >= 0:
                        sca[j - LOOK].wait()
                    elif not first:
                        wait_scatter(t % NBUF)
                    gat[t] = do_gather(t, t % NBUF, sb)
                else:
                    sca[j - LOOK].wait()
                    if t == IDX_G:
                        d_a.wait()
                        d_b.wait()
                    tn = t - IDX_G
                    do_gather(tn, tn % NBUF, nsb)
                if j in gat:
                    gat[j].wait()
                else:
                    wait_gather(b)
                sca[j] = do_scatter(j, b)

        group_body(0, True)

        def group(gi, carry):
            group_body(gi, False)
            return carry

        lax.fori_loop(1, IDX_GROUPS, group, 0)
        for j in range(LOOK):
            wait_gather(j)
        for j in range(IDX_G - LOOK, IDX_G):
            wait_scatter(j % NBUF)

    @pl.when(cid == 0)
    def _():
        edges(g0_hbm)

    @pl.when(cid == 1)
    def _():
        edges(g1_hbm)

    plsc.subcore_barrier()

    def out(out_hbm):
        sl = pl.ds(sid * TILE_ROWS, TILE_ROWS)
        pltpu.sync_copy(acc.at[sl], out_hbm.at[sl])

    @pl.when(cid == 0)
    def _():
        out(out0_hbm)

    @pl.when(cid == 1)
    def _():
        out(out1_hbm)


def _conv_call(g0, g1, src_c, dst_c):
    return pl.kernel(
        _conv_body,
        out_type=(
            jax.ShapeDtypeStruct((NROW, HHALF), jnp.float32),
            jax.ShapeDtypeStruct((NROW, HHALF), jnp.float32),
        ),
        mesh=_mesh,
        compiler_params=_sc_params,
        scratch_types=[
            pltpu.VMEM_SHARED((NROW, HHALF), jnp.float32),
            pltpu.VMEM((2, IDX_G, CHUNK), jnp.int32),
            pltpu.VMEM((2, IDX_G, CHUNK), jnp.int32),
            pltpu.VMEM((NBUF, CHUNK, HHALF), jnp.float32),
        ] + [pltpu.SemaphoreType.DMA] * 9,
    )(g0, g1, src_c, dst_c)


NROW4 = NROW * HHALF // 128
RP = 256
NBLK = -(-NROW4 // RP)


def _dot(a, b):
    return lax.dot_general(a, b, (((1,), (0,)), ((), ())),
                           precision=lax.Precision.HIGHEST,
                           preferred_element_type=jnp.float32)


def _prep_body(x4_ref, dg0_ref, dg1_ref, w0_ref, w1_ref, g0_ref, g1_ref, dv_ref):
    dv = lax.rsqrt(dg0_ref[...] + dg1_ref[...] + 1.0)
    dv_ref[...] = dv
    g0_ref[...] = _dot(x4_ref[...], w0_ref[...]) * dv
    g1_ref[...] = _dot(x4_ref[...], w1_ref[...]) * dv


def _prep_call(x4, dg0, dg1, w1bd0, w1bd1):
    return pl.pallas_call(
        _prep_body,
        grid=(NBLK,),
        in_specs=[
            pl.BlockSpec((RP, 4 * CIN), lambda i: (i, 0)),
            pl.BlockSpec((RP, 128), lambda i: (i, 0)),
            pl.BlockSpec((RP, 128), lambda i: (i, 0)),
            pl.BlockSpec((4 * CIN, 128), lambda i: (0, 0)),
            pl.BlockSpec((4 * CIN, 128), lambda i: (0, 0)),
        ],
        out_specs=[pl.BlockSpec((RP, 128), lambda i: (i, 0))] * 3,
        out_shape=[jax.ShapeDtypeStruct((NROW4, 128), jnp.float32)] * 3,
    )(x4, dg0, dg1, w1bd0, w1bd1)


def _mid_body(a0_ref, a1_ref, dv_ref, b0_ref, b1_ref,
              w00_ref, w10_ref, w01_ref, w11_ref, g0_ref, g1_ref):
    dv = dv_ref[...]
    h0 = jnp.maximum(a0_ref[...] * dv + b0_ref[...], 0.0)
    h1 = jnp.maximum(a1_ref[...] * dv + b1_ref[...], 0.0)
    g0_ref[...] = (_dot(h0, w00_ref[...]) + _dot(h1, w10_ref[...])) * dv
    g1_ref[...] = (_dot(h0, w01_ref[...]) + _dot(h1, w11_ref[...])) * dv


def _mid_call(a0p, a1p, dinvp, b0p, b1p, wbds):
    return pl.pallas_call(
        _mid_body,
        grid=(NBLK,),
        in_specs=[
            pl.BlockSpec((RP, 128), lambda i: (i, 0)),
            pl.BlockSpec((RP, 128), lambda i: (i, 0)),
            pl.BlockSpec((RP, 128), lambda i: (i, 0)),
            pl.BlockSpec((1, 128), lambda i: (0, 0)),
            pl.BlockSpec((1, 128), lambda i: (0, 0)),
        ] + [pl.BlockSpec((128, 128), lambda i: (0, 0))] * 4,
        out_specs=[pl.BlockSpec((RP, 128), lambda i: (i, 0))] * 2,
        out_shape=[jax.ShapeDtypeStruct((NROW4, 128), jnp.float32)] * 2,
    )(a0p, a1p, dinvp, b0p, b1p, *wbds)


def _final_body(a0_ref, a1_ref, dv_ref, b0_ref, b1_ref, batchq_ref,
                wf1_ref, bf1_ref, wf2_ref, bf2_ref, out_ref, pooled, cnt):
    i = pl.program_id(0)

    @pl.when(i == 0)
    def _():
        pooled[...] = jnp.zeros_like(pooled)
        cnt[...] = jnp.zeros_like(cnt)

    dv = dv_ref[...]
    h0 = jnp.maximum(a0_ref[...] * dv + b0_ref[...], 0.0)
    h1 = jnp.maximum(a1_ref[...] * dv + b1_ref[...], 0.0)
    rid = i * RP + lax.broadcasted_iota(jnp.int32, (RP, 1), 0)
    ok = rid < NROW4
    h0 = jnp.where(ok, h0, 0.0)
    h1 = jnp.where(ok, h1, 0.0)
    p0 = jnp.zeros((NGRAPH, HHALF), jnp.float32)
    p1 = jnp.zeros((NGRAPH, HHALF), jnp.float32)
    c8 = jnp.zeros((NGRAPH, 8), jnp.float32)
    for q in range(4):
        bq = batchq_ref[0, q, :]
        oh = (lax.broadcasted_iota(jnp.int32, (NGRAPH, RP), 0)
              == bq[None, :]).astype(jnp.float32)
        p0 += _dot(oh, h0[:, 32 * q:32 * q + 32])
        p1 += _dot(oh, h1[:, 32 * q:32 * q + 32])
        c8 += _dot(oh, jnp.ones((RP, 8), jnp.float32))
    pooled[...] += jnp.concatenate([p0, p1], axis=1)
    cnt[...] += c8

    @pl.when(i == NBLK - 1)
    def _():
        pm = pooled[...] / jnp.maximum(cnt[...][:, :1], 1.0)
        z = jnp.maximum(_dot(pm, wf1_ref[...]) + bf1_ref[...], 0.0)
        out_ref[...] = _dot(z, wf2_ref[...]) + bf2_ref[...]


def _final_call(a0p, a1p, dinvp, b0p, b1p, batchq, Wf1, bf1r, Wf2p, bf2p):
    return pl.pallas_call(
        _final_body,
        grid=(NBLK,),
        in_specs=[
            pl.BlockSpec((RP, 128), lambda i: (i, 0)),
            pl.BlockSpec((RP, 128), lambda i: (i, 0)),
            pl.BlockSpec((RP, 128), lambda i: (i, 0)),
            pl.BlockSpec((1, 128), lambda i: (0, 0)),
            pl.BlockSpec((1, 128), lambda i: (0, 0)),
            pl.BlockSpec((1, 4, RP), lambda i: (i, 0, 0)),
            pl.BlockSpec((HID, HID), lambda i: (0, 0)),
            pl.BlockSpec((1, HID), lambda i: (0, 0)),
            pl.BlockSpec((HID, 128), lambda i: (0, 0)),
            pl.BlockSpec((1, 128), lambda i: (0, 0)),
        ],
        out_specs=pl.BlockSpec((NGRAPH, 128), lambda i: (0, 0)),
        out_shape=jax.ShapeDtypeStruct((NGRAPH, 128), jnp.float32),
        scratch_shapes=[
            pltpu.VMEM((NGRAPH, HID), jnp.float32),
            pltpu.VMEM((NGRAPH, 8), jnp.float32),
        ],
    )(a0p, a1p, dinvp, b0p, b1p, batchq, Wf1, bf1r, Wf2p, bf2p)


def _bd4(m):
    return jnp.kron(jnp.eye(4, dtype=m.dtype), m)


def _packb(b_half):
    return jnp.tile(b_half, 4).reshape(1, 128)


def kernel(x, edge_index, batch, W1, b1, W2, b2, W3, b3, Wf1, bf1, Wf2, bf2):
    i32 = jnp.int32
    src = edge_index[0]
    dst = edge_index[1]
    pad_src = (jnp.arange(E_CONV - E, dtype=i32) * 641) % N
    pad_dst = SINK + jnp.arange(E_CONV - E, dtype=i32) % (NROW - N)
    src_c = jnp.concatenate([src, pad_src]).reshape(NS, CONV_CHUNKS, CHUNK)
    dst_c = jnp.concatenate([dst, pad_dst]).reshape(NS, CONV_CHUNKS, CHUNK)
    pad_dd = SINK + jnp.arange(E_DEG - E, dtype=i32) % (NROW - N)
    dst_d = jnp.concatenate([dst, pad_dd]).reshape(NC * NS, DEG_CHUNKS, CHUNK)
    zeros_init = jnp.zeros((TILE_ROWS, HHALF), jnp.float32)
    ones_rows = jnp.ones((CHUNK, HHALF), jnp.float32)
    x4 = jnp.pad(x, ((0, NROW - N), (0, 0))).reshape(NROW4, 4 * CIN)
    batchq = (jnp.concatenate([batch, jnp.full((NBLK * 4 * RP - N,), -1, i32)])
              .reshape(NBLK, RP, 4).transpose(0, 2, 1))
    w1bd0 = _bd4(W1[:, :HHALF])
    w1bd1 = _bd4(W1[:, HHALF:])

    def mid_wbds(W):
        return (_bd4(W[:HHALF, :HHALF]), _bd4(W[HHALF:, :HHALF]),
                _bd4(W[:HHALF, HHALF:]), _bd4(W[HHALF:, HHALF:]))

    bf1r = bf1.reshape(1, HID)
    Wf2p = jnp.pad(Wf2, ((0, 0), (0, 128 - NCLS)))
    bf2p = jnp.pad(bf2, (0, 128 - NCLS)).reshape(1, 128)

    dg0, dg1 = _deg_call(dst_d, zeros_init, ones_rows)
    g0p, g1p, dinvp = _prep_call(x4, dg0.reshape(NROW4, 128),
                                 dg1.reshape(NROW4, 128), w1bd0, w1bd1)

    def conv(g0p_, g1p_):
        a0, a1 = _conv_call(g0p_.reshape(NROW, HHALF), g1p_.reshape(NROW, HHALF),
                            src_c, dst_c)
        return a0.reshape(NROW4, 128), a1.reshape(NROW4, 128)

    a0p, a1p = conv(g0p, g1p)
    g0p, g1p = _mid_call(a0p, a1p, dinvp, _packb(b1[:HHALF]), _packb(b1[HHALF:]),
                         mid_wbds(W2))
    a0p, a1p = conv(g0p, g1p)
    g0p, g1p = _mid_call(a0p, a1p, dinvp, _packb(b2[:HHALF]), _packb(b2[HHALF:]),
                         mid_wbds(W3))
    a0p, a1p = conv(g0p, g1p)
    outp = _final_call(a0p, a1p, dinvp, _packb(b3[:HHALF]), _packb(b3[HHALF:]),
                       batchq, Wf1, bf1r, Wf2p, bf2p)
    return outp[:, :NCLS]
